# Initial kernel scaffold; baseline (speedup 1.0000x reference)
import jax
import jax.numpy as jnp
from jax.experimental import pallas as pl


def kernel(x, edge_index, edge_attr, W1, root1, bias1, W2, root2, bias2):
    raise NotImplementedError("write your pallas kernel here")


# speedup vs baseline: 24.7016x; 24.7016x over previous
import functools

import jax
import jax.numpy as jnp
from jax import lax
from jax.experimental import pallas as pl
from jax.experimental.pallas import tpu as pltpu
from jax.experimental.pallas import tpu_sc as plsc

N = 10000
E = 160000
D = 128
K9 = 9
DY = K9 * D

NC, NS = 2, 16
CH = 64
NCHUNK = E // CH
ITERS = -(-NCHUNK // NS)
NSPL = N // NC
NPL = 5120
TRASH = NSPL
RPT = NPL // NS

BM = 1000


def _Z():
    return jnp.int32(0)


def _mm_body(x_ref, w_ref, b_ref, y_ref, r_ref):
    acc = jnp.dot(x_ref[...], w_ref[...], preferred_element_type=jnp.float32)
    y_ref[...] = acc[:, :DY]
    r_ref[...] = acc[:, DY:] + b_ref[...]


def _mm_call(x, wcat, bias_row):
    return pl.pallas_call(
        _mm_body,
        grid=(N // BM,),
        in_specs=[
            pl.BlockSpec((BM, D), lambda i: (i, _Z())),
            pl.BlockSpec((D, DY + D), lambda i: (_Z(), _Z())),
            pl.BlockSpec((1, D), lambda i: (_Z(), _Z())),
        ],
        out_specs=[
            pl.BlockSpec((BM, DY), lambda i: (i, _Z())),
            pl.BlockSpec((BM, D), lambda i: (i, _Z())),
        ],
        out_shape=[
            jax.ShapeDtypeStruct((N, DY), jnp.float32),
            jax.ShapeDtypeStruct((N, D), jnp.float32),
        ],
    )(x, wcat, bias_row)


def _comb_body(p_ref, r_ref, o_ref):
    o_ref[...] = jnp.maximum(p_ref[0] + r_ref[...], 0.0)


def _comb_call(partials, r):
    return pl.pallas_call(
        _comb_body,
        grid=(N // BM,),
        in_specs=[
            pl.BlockSpec((1, BM, D),
                         lambda i: (i // (NSPL // BM),
                                    i % (NSPL // BM), _Z())),
            pl.BlockSpec((BM, D), lambda i: (i, _Z())),
        ],
        out_specs=pl.BlockSpec((BM, D), lambda i: (i, _Z())),
        out_shape=jax.ShapeDtypeStruct((N, D), jnp.float32),
    )(partials, r)


def _frac(v):
    return v - lax.convert_element_type(
        lax.convert_element_type(v, jnp.int32), jnp.float32)


def _basis3(f):
    ff = f * f
    return (0.5 * ff - f + 0.5, -ff + f + 0.5, 0.5 * ff)


def _sc_body(y_hbm, src_hbm, dst_hbm, a0_hbm, a1_hbm, zero_hbm,
             out_hbm, src_v, idx_v, a0_v, a1_v, rows_v, msg_v, acc_sh, sem):
    i32 = jnp.int32
    c = lax.axis_index("c")
    s = lax.axis_index("s")
    row0 = s * i32(RPT)
    nbase = c * i32(NSPL)

    pltpu.sync_copy(zero_hbm.at[pl.ds(row0, RPT)],
                    acc_sh.at[pl.ds(row0, RPT)])
    plsc.subcore_barrier()

    def chunk_body(i, _):
        cid = i * i32(NS) + s

        @pl.when(cid < i32(NCHUNK))
        def _():
            eb = cid * i32(CH)
            pltpu.sync_copy(src_hbm.at[pl.ds(eb, CH)], src_v)
            pltpu.sync_copy(dst_hbm.at[pl.ds(eb, CH)], idx_v)
            pltpu.sync_copy(a0_hbm.at[pl.ds(eb, CH)], a0_v)
            pltpu.sync_copy(a1_hbm.at[pl.ds(eb, CH)], a1_v)
            pltpu.async_copy(y_hbm.at[src_v], rows_v, sem).wait()

            for g in range(CH // 16):
                f0 = _frac(a0_v[pl.ds(g * 16, 16)])
                f1 = _frac(a1_v[pl.ds(g * 16, 16)])
                b0 = _basis3(f0)
                b1 = _basis3(f1)
                cs = [b0[k0] * b1[k1]
                      for k0 in range(3) for k1 in range(3)]

                def edge_body(t, _):
                    e = i32(g * 16) + t
                    lane = jnp.full((16,), t, jnp.int32)
                    bks = [jnp.take(ck, lane) for ck in cs]
                    for j in range(D // 16):
                        acc = None
                        for k in range(K9):
                            seg = rows_v[e, pl.ds(k * D + j * 16, 16)]
                            term = seg * bks[k]
                            acc = term if acc is None else acc + term
                        msg_v[e, pl.ds(j * 16, 16)] = acc
                    return i32(0)

                lax.fori_loop(i32(0), i32(16), edge_body, i32(0))

                dstg = idx_v[pl.ds(g * 16, 16)] - nbase
                ok = (dstg >= i32(0)) & (dstg < i32(NSPL))
                idx_v[pl.ds(g * 16, 16)] = jnp.where(ok, dstg, i32(TRASH))

            pltpu.sync_copy(msg_v, acc_sh.at[idx_v], add=True)

        return i32(0)

    lax.fori_loop(i32(0), i32(ITERS), chunk_body, i32(0))
    plsc.subcore_barrier()

    pltpu.sync_copy(acc_sh.at[pl.ds(row0, RPT)],
                    out_hbm.at[c, pl.ds(row0, RPT)])


@functools.cache
def _sc_aggregate():
    mesh = plsc.VectorSubcoreMesh(
        core_axis_name="c", subcore_axis_name="s",
        num_cores=NC, num_subcores=NS)
    return pl.kernel(
        _sc_body,
        mesh=mesh,
        out_type=jax.ShapeDtypeStruct((NC, NPL, D), jnp.float32),
        scratch_types=[
            pltpu.VMEM((CH,), jnp.int32),
            pltpu.VMEM((CH,), jnp.int32),
            pltpu.VMEM((CH,), jnp.float32),
            pltpu.VMEM((CH,), jnp.float32),
            pltpu.VMEM((CH, DY), jnp.float32),
            pltpu.VMEM((CH, D), jnp.float32),
            pltpu.VMEM_SHARED((NPL, D), jnp.float32),
            pltpu.SemaphoreType.DMA,
        ],
    )


def _layer(x, src, dst, a0, a1, W, root, bias, zero_nd):
    wcat = jnp.concatenate(
        [jnp.transpose(W, (1, 0, 2)).reshape(D, DY), root], axis=1)
    y, r = _mm_call(x, wcat, bias.reshape(1, D))
    partials = _sc_aggregate()(y, src, dst, a0, a1, zero_nd)
    return _comb_call(partials, r)


def kernel(x, edge_index, edge_attr, W1, root1, bias1, W2, root2, bias2):
    x = x.astype(jnp.float32)
    src = edge_index[0].astype(jnp.int32)
    dst = edge_index[1].astype(jnp.int32)
    a0 = edge_attr[:, 0].astype(jnp.float32)
    a1 = edge_attr[:, 1].astype(jnp.float32)
    zero_nd = jnp.zeros((NPL, D), jnp.float32)
    h = _layer(x, src, dst, a0, a1, W1.astype(jnp.float32),
               root1.astype(jnp.float32), bias1.astype(jnp.float32), zero_nd)
    h2 = _layer(h, src, dst, a0, a1, W2.astype(jnp.float32),
                root2.astype(jnp.float32), bias2.astype(jnp.float32), zero_nd)
    return h2.astype(jnp.float64)


# speedup vs baseline: 36.9712x; 1.4967x over previous
import functools

import jax
import jax.numpy as jnp
from jax import lax
from jax.experimental import pallas as pl
from jax.experimental.pallas import tpu as pltpu
from jax.experimental.pallas import tpu_sc as plsc

N = 10000
E = 160000
D = 128
K9 = 9
DY = K9 * D

NC, NS = 2, 16
CH = 32
NCHUNK = E // CH
CPS = -(-NCHUNK // NS)
CB = 32
NBLK = -(-CPS // CB)
ECB = CB * CH
EPAD = ((NS - 1) * CPS + NBLK * CB) * CH
NSPL = N // NC
NPL = 5120
TRASH = NSPL
RPT = NPL // NS

BM = 1000


def _Z():
    return jnp.int32(0)


def _mm_body(x_ref, w_ref, b_ref, y_ref, r_ref):
    acc = jnp.dot(x_ref[...], w_ref[...], preferred_element_type=jnp.float32)
    y_ref[...] = acc[:, :DY]
    r_ref[...] = acc[:, DY:] + b_ref[...]


def _mm_call(x, wcat, bias_row):
    return pl.pallas_call(
        _mm_body,
        grid=(N // BM,),
        in_specs=[
            pl.BlockSpec((BM, D), lambda i: (i, _Z())),
            pl.BlockSpec((D, DY + D), lambda i: (_Z(), _Z())),
            pl.BlockSpec((1, D), lambda i: (_Z(), _Z())),
        ],
        out_specs=[
            pl.BlockSpec((BM, DY), lambda i: (i, _Z())),
            pl.BlockSpec((BM, D), lambda i: (i, _Z())),
        ],
        out_shape=[
            jax.ShapeDtypeStruct((N, DY), jnp.float32),
            jax.ShapeDtypeStruct((N, D), jnp.float32),
        ],
    )(x, wcat, bias_row)


def _comb_body(p_ref, r_ref, o_ref):
    o_ref[...] = jnp.maximum(p_ref[0] + r_ref[...], 0.0)


def _comb_call(partials, r):
    return pl.pallas_call(
        _comb_body,
        grid=(N // BM,),
        in_specs=[
            pl.BlockSpec((1, BM, D),
                         lambda i: (i // (NSPL // BM),
                                    i % (NSPL // BM), _Z())),
            pl.BlockSpec((BM, D), lambda i: (i, _Z())),
        ],
        out_specs=pl.BlockSpec((BM, D), lambda i: (i, _Z())),
        out_shape=jax.ShapeDtypeStruct((N, D), jnp.float32),
    )(partials, r)


def _frac(v):
    return v - lax.convert_element_type(
        lax.convert_element_type(v, jnp.int32), jnp.float32)


def _basis3(f):
    ff = f * f
    return (0.5 * ff - f + 0.5, -ff + f + 0.5, 0.5 * ff)


def _sc_body(y_hbm, src_hbm, dst_hbm, a0_hbm, a1_hbm, zero_hbm,
             out_hbm, src_b, dst_b, a0_b, a1_b, idx_v,
             rows0_v, rows1_v, msg_v, acc_sh, sem0, sem1):
    i32 = jnp.int32
    c = lax.axis_index("c")
    s = lax.axis_index("s")
    row0 = s * i32(RPT)
    nbase = c * i32(NSPL)

    pltpu.sync_copy(zero_hbm.at[pl.ds(row0, RPT)],
                    acc_sh.at[pl.ds(row0, RPT)])
    plsc.subcore_barrier()

    def block_body(b, _):
        tb = b * i32(CB)
        eb = (s * i32(CPS) + tb) * i32(CH)
        pltpu.sync_copy(src_hbm.at[pl.ds(eb, ECB)], src_b)
        pltpu.sync_copy(dst_hbm.at[pl.ds(eb, ECB)], dst_b)
        pltpu.sync_copy(a0_hbm.at[pl.ds(eb, ECB)], a0_b)
        pltpu.sync_copy(a1_hbm.at[pl.ds(eb, ECB)], a1_b)

        def valid(k):
            t = tb + k
            return (t < i32(CPS)) & (s * i32(CPS) + t < i32(NCHUNK))

        def fire(k, rows_v, sem):
            @pl.when(valid(k) & (k < i32(CB)))
            def _():
                lo = k * i32(CH)
                pltpu.async_copy(y_hbm.at[src_b.at[pl.ds(lo, CH)]],
                                 rows_v, sem)

        def consume(k, rows_v, sem):
            @pl.when(valid(k))
            def _():
                pltpu.make_async_copy(y_hbm.at[src_b.at[pl.ds(i32(0), CH)]],
                                      rows_v, sem).wait()
                lo = k * i32(CH)
                for g in range(CH // 16):
                    f0 = _frac(a0_b[pl.ds(lo + i32(g * 16), 16)])
                    f1 = _frac(a1_b[pl.ds(lo + i32(g * 16), 16)])
                    b0 = _basis3(f0)
                    b1 = _basis3(f1)
                    cs = [b0[k0] * b1[k1]
                          for k0 in range(3) for k1 in range(3)]

                    def edge_body(t, _):
                        e = i32(g * 16) + t
                        lane = jnp.full((16,), t, jnp.int32)
                        bks = [jnp.take(ck, lane) for ck in cs]
                        for j in range(D // 16):
                            acc = None
                            for k9 in range(K9):
                                seg = rows_v[e, pl.ds(k9 * D + j * 16, 16)]
                                term = seg * bks[k9]
                                acc = term if acc is None else acc + term
                            msg_v[e, pl.ds(j * 16, 16)] = acc
                        return i32(0)

                    lax.fori_loop(i32(0), i32(16), edge_body, i32(0))

                    dstg = dst_b[pl.ds(lo + i32(g * 16), 16)] - nbase
                    ok = (dstg >= i32(0)) & (dstg < i32(NSPL))
                    idx_v[pl.ds(g * 16, 16)] = jnp.where(ok, dstg, i32(TRASH))

                pltpu.sync_copy(msg_v, acc_sh.at[idx_v], add=True)

        fire(i32(0), rows0_v, sem0)

        def pair_body(p, _):
            k = p * i32(2)
            fire(k + i32(1), rows1_v, sem1)
            consume(k, rows0_v, sem0)
            fire(k + i32(2), rows0_v, sem0)
            consume(k + i32(1), rows1_v, sem1)
            return i32(0)

        lax.fori_loop(i32(0), i32(CB // 2), pair_body, i32(0))
        return i32(0)

    lax.fori_loop(i32(0), i32(NBLK), block_body, i32(0))
    plsc.subcore_barrier()

    pltpu.sync_copy(acc_sh.at[pl.ds(row0, RPT)],
                    out_hbm.at[c, pl.ds(row0, RPT)])


@functools.cache
def _sc_aggregate():
    mesh = plsc.VectorSubcoreMesh(
        core_axis_name="c", subcore_axis_name="s",
        num_cores=NC, num_subcores=NS)
    return pl.kernel(
        _sc_body,
        mesh=mesh,
        out_type=jax.ShapeDtypeStruct((NC, NPL, D), jnp.float32),
        scratch_types=[
            pltpu.VMEM((ECB,), jnp.int32),
            pltpu.VMEM((ECB,), jnp.int32),
            pltpu.VMEM((ECB,), jnp.float32),
            pltpu.VMEM((ECB,), jnp.float32),
            pltpu.VMEM((CH,), jnp.int32),
            pltpu.VMEM((CH, DY), jnp.float32),
            pltpu.VMEM((CH, DY), jnp.float32),
            pltpu.VMEM((CH, D), jnp.float32),
            pltpu.VMEM_SHARED((NPL, D), jnp.float32),
            pltpu.SemaphoreType.DMA,
            pltpu.SemaphoreType.DMA,
        ],
    )


def _layer(x, src, dst, a0, a1, W, root, bias, zero_nd):
    wcat = jnp.concatenate(
        [jnp.transpose(W, (1, 0, 2)).reshape(D, DY), root], axis=1)
    y, r = _mm_call(x, wcat, bias.reshape(1, D))
    partials = _sc_aggregate()(y, src, dst, a0, a1, zero_nd)
    return _comb_call(partials, r)


def kernel(x, edge_index, edge_attr, W1, root1, bias1, W2, root2, bias2):
    x = x.astype(jnp.float32)
    padn = (0, EPAD - E)
    src = jnp.pad(edge_index[0].astype(jnp.int32), padn)
    dst = jnp.pad(edge_index[1].astype(jnp.int32), padn)
    a0 = jnp.pad(edge_attr[:, 0].astype(jnp.float32), padn)
    a1 = jnp.pad(edge_attr[:, 1].astype(jnp.float32), padn)
    zero_nd = jnp.zeros((NPL, D), jnp.float32)
    h = _layer(x, src, dst, a0, a1, W1.astype(jnp.float32),
               root1.astype(jnp.float32), bias1.astype(jnp.float32), zero_nd)
    h2 = _layer(h, src, dst, a0, a1, W2.astype(jnp.float32),
                root2.astype(jnp.float32), bias2.astype(jnp.float32), zero_nd)
    return h2.astype(jnp.float64)
